# Initial kernel scaffold; baseline (speedup 1.0000x reference)
import jax
import jax.numpy as jnp
from jax.experimental import pallas as pl


def kernel(x, r_dir, tables, d_w1, d_b1, d_w2, d_b2, c_w1, c_b1, c_w2, c_b2, c_w3, c_b3):
    raise NotImplementedError("write your pallas kernel here")


# speedup vs baseline: 5.7827x; 5.7827x over previous
import functools
import numpy as np
import jax
import jax.numpy as jnp
from jax import lax
from jax.experimental import pallas as pl
from jax.experimental.pallas import tpu as pltpu
from jax.experimental.pallas import tpu_sc as plsc

_SCENE_SCALE = 0.5
_T = 2 ** 19
_NLEVELS = 16
_F = 2
_L = 4
_LEVELS = np.geomspace(16, 2048, _NLEVELS, dtype=int)
_PR = [1, -1640531535, 805459861]

_NW = 32
_C = 512


def _sc_features_body(xsx_hbm, xsy_hbm, xsz_hbm, table_hbm, out_hbm,
                      xb0, xb1, xb2, idx0, idx1, rows0, rows1, feat, sem0, sem1):
    npts = out_hbm.shape[0]
    ppw = npts // _NW
    nchunks = ppw // _C
    wid = lax.axis_index("s") * jnp.int32(2) + lax.axis_index("c")

    lane = lax.iota(jnp.int32, 16)
    hc = 8 * _C

    xbufs = [xb0, xb1, xb2]
    idxbufs = [idx0, idx1]
    rowbufs = [rows0, rows1]
    sems = [sem0, sem1]

    def chunk_body(c, carry):
        base = wid * jnp.int32(ppw) + c * jnp.int32(_C)
        for d, src in enumerate((xsx_hbm, xsy_hbm, xsz_hbm)):
            pltpu.sync_copy(src.at[pl.ds(base, _C)], xbufs[d])

        def build_idx(l, ibuf):
            scale = jnp.float32(float(_LEVELS[l]))
            lbase = jnp.int32(2 * l * _T)

            def body_a(i, carry_a):
                off = i * jnp.int32(16)
                xv = [xbufs[d][pl.ds(off, 16)] for d in range(3)]
                y = []
                for d in range(3):
                    p = xv[d] * scale
                    fi = p.astype(jnp.int32)
                    ylo = fi * jnp.int32(_PR[d]) if d else fi
                    y.append((ylo, ylo + jnp.int32(_PR[d])))
                for k in range(8):
                    z = y[0][k & 1] ^ y[1][(k >> 1) & 1] ^ y[2][(k >> 2) & 1]
                    e0 = ((z & jnp.int32(_T - 1)) << 1) | lbase
                    ibuf[pl.ds(jnp.int32(k * _C) + off, 16)] = e0
                    ibuf[pl.ds(jnp.int32(hc + k * _C) + off, 16)] = e0 | jnp.int32(1)
                return carry_a

            lax.fori_loop(jnp.int32(0), jnp.int32(_C // 16), body_a, 0)

        def accum(l, rbuf):
            scale = jnp.float32(float(_LEVELS[l]))
            col0 = jnp.zeros((16,), jnp.int32) + jnp.int32(2 * l)
            col1 = col0 + jnp.int32(1)

            def body_b(i, carry_b):
                off = i * jnp.int32(16)
                pidx = off + lane
                t = []
                u = []
                for d in range(3):
                    p = xbufs[d][pl.ds(off, 16)] * scale
                    fl = p.astype(jnp.int32).astype(jnp.float32)
                    td = p - fl
                    t.append(td)
                    u.append(1.0 - td)
                pxy = [u[0] * u[1], t[0] * u[1], u[0] * t[1], t[0] * t[1]]
                acc0 = jnp.zeros((16,), jnp.float32)
                acc1 = jnp.zeros((16,), jnp.float32)
                for k in range(8):
                    w = pxy[k & 3] * (t[2] if k & 4 else u[2])
                    rv0 = rbuf[pl.ds(jnp.int32(k * _C) + off, 16)]
                    rv1 = rbuf[pl.ds(jnp.int32(hc + k * _C) + off, 16)]
                    acc0 = acc0 + rv0 * w
                    acc1 = acc1 + rv1 * w
                plsc.store_scatter(feat, [pidx, col0], acc0)
                plsc.store_scatter(feat, [pidx, col1], acc1)
                return carry_b

            lax.fori_loop(jnp.int32(0), jnp.int32(_C // 16), body_b, 0)

        descs = [None, None]
        for l in range(_NLEVELS):
            b = l & 1
            build_idx(l, idxbufs[b])
            descs[b] = pltpu.async_copy(table_hbm.at[idxbufs[b]], rowbufs[b], sems[b])
            if l > 0:
                descs[1 - b].wait()
                accum(l - 1, rowbufs[1 - b])
        descs[1].wait()
        accum(_NLEVELS - 1, rowbufs[1])

        pltpu.sync_copy(feat, out_hbm.at[pl.ds(base, _C)])
        return carry

    lax.fori_loop(jnp.int32(0), jnp.int32(nchunks), chunk_body, 0)


def _sc_features(xs3, table_flat, npts):
    mesh = plsc.VectorSubcoreMesh(core_axis_name="c", subcore_axis_name="s")
    kfn = pl.kernel(
        _sc_features_body,
        out_type=jax.ShapeDtypeStruct((npts, _NLEVELS * _F), jnp.float32),
        mesh=mesh,
        scratch_types=[
            pltpu.VMEM((_C,), jnp.float32),
            pltpu.VMEM((_C,), jnp.float32),
            pltpu.VMEM((_C,), jnp.float32),
            pltpu.VMEM((16 * _C,), jnp.int32),
            pltpu.VMEM((16 * _C,), jnp.int32),
            pltpu.VMEM((16 * _C,), jnp.float32),
            pltpu.VMEM((16 * _C,), jnp.float32),
            pltpu.VMEM((_C, _NLEVELS * _F), jnp.float32),
            pltpu.SemaphoreType.DMA,
            pltpu.SemaphoreType.DMA,
        ],
        compiler_params=pltpu.CompilerParams(
            needs_layout_passes=False, use_tc_tiling_on_sc=False),
    )
    return kfn(xs3[0], xs3[1], xs3[2], table_flat)


def _tc_mlp_body(featT_ref, aux_ref,
                 dw1_ref, db1_ref, dw2_ref, db2_ref,
                 wls_ref, wrd_ref, wenc_ref, cb1_ref,
                 cw2_ref, cb2_ref, cw3_ref, cb3_ref,
                 m_ref, mb_ref,
                 out_ref):
    hp = jax.lax.Precision.HIGHEST
    fT = featT_ref[...]
    aux = aux_ref[...]
    rd = aux[0:3]
    h1 = jnp.maximum(jnp.dot(dw1_ref[...], fT, precision=hp) + db1_ref[...], 0.0)
    ls = jnp.dot(dw2_ref[...], h1, precision=hp) + db2_ref[...]
    enc = jnp.sin(jnp.dot(m_ref[...], rd, precision=hp) + mb_ref[...])
    h = (jnp.dot(wls_ref[...], ls, precision=hp)
         + jnp.dot(wrd_ref[...], rd, precision=hp)
         + jnp.dot(wenc_ref[...], enc, precision=hp)
         + cb1_ref[...])
    h = jnp.maximum(h, 0.0)
    h = jnp.maximum(jnp.dot(cw2_ref[...], h, precision=hp) + cb2_ref[...], 0.0)
    o = jax.nn.sigmoid(jnp.dot(cw3_ref[...], h, precision=hp) + cb3_ref[...])
    xs = aux[3:6]
    inb = (xs > 0.0) & (xs < 1.0)
    m = (inb[0:1] & inb[1:2] & inb[2:3]).astype(jnp.float32)
    sig = jnp.exp(ls[0:1]) * m
    out_ref[...] = jnp.concatenate(
        [o * m, sig, jnp.zeros_like(aux[0:4])], axis=0)


def _tc_mlp(featT, aux8, weights, npts):
    blk = 8192
    grid = (npts // blk,)

    def col_spec(nrow):
        return pl.BlockSpec((nrow, blk), lambda i: (i - i, i))

    def full_spec(a):
        return pl.BlockSpec(a.shape, lambda i, n=a.ndim: (i - i,) * n)

    in_specs = [col_spec(_NLEVELS * _F), col_spec(8)]
    in_specs += [full_spec(w) for w in weights]
    out_specs = col_spec(8)
    out_shape = jax.ShapeDtypeStruct((8, npts), jnp.float32)
    return pl.pallas_call(
        _tc_mlp_body,
        grid=grid,
        in_specs=in_specs,
        out_specs=out_specs,
        out_shape=out_shape,
    )(featT, aux8, *weights)


def kernel(x, r_dir, tables, d_w1, d_b1, d_w2, d_b2,
           c_w1, c_b1, c_w2, c_b2, c_w3, c_b3):
    N, S, _ = x.shape
    npts = N * S

    xs_flat = (x.reshape(npts, 3) * _SCENE_SCALE + 0.5).astype(jnp.float32)
    rd_flat = r_dir.reshape(npts, 3).astype(jnp.float32)
    xs3 = xs_flat.T
    table_flat = tables.reshape(_NLEVELS * _T * _F).astype(jnp.float32)

    feat = _sc_features(xs3, table_flat, npts)

    freqs = 2.0 ** np.arange(_L)
    M = np.zeros((3, 3 * 2 * _L), np.float32)
    Mb = np.zeros((1, 3 * 2 * _L), np.float32)
    for cdim in range(3):
        for j in range(_L):
            M[cdim, cdim * 2 * _L + j] = 2.0 * np.pi * freqs[j]
            M[cdim, cdim * 2 * _L + _L + j] = 2.0 * np.pi * freqs[j]
            Mb[0, cdim * 2 * _L + _L + j] = np.pi / 2.0
    f32 = jnp.float32
    weights = [
        d_w1.T.astype(f32), d_b1.reshape(-1, 1).astype(f32),
        d_w2.T.astype(f32), d_b2.reshape(-1, 1).astype(f32),
        c_w1[:16].T.astype(f32), c_w1[16:19].T.astype(f32),
        c_w1[19:].T.astype(f32), c_b1.reshape(-1, 1).astype(f32),
        c_w2.T.astype(f32), c_b2.reshape(-1, 1).astype(f32),
        c_w3.T.astype(f32), c_b3.reshape(-1, 1).astype(f32),
        jnp.asarray(M.T), jnp.asarray(Mb.reshape(-1, 1)),
    ]
    featT = feat.T
    aux8 = jnp.concatenate(
        [rd_flat.T, xs3, jnp.zeros((2, npts), jnp.float32)], axis=0)
    out8 = _tc_mlp(featT, aux8, weights, npts)
    color = out8[0:3].T
    sigma = out8[3:4].T
    return (color.reshape(N, S, 3).astype(jnp.float64),
            sigma.reshape(N, S, 1).astype(jnp.float64))
